# Initial kernel scaffold; baseline (speedup 1.0000x reference)
import functools

import jax
import jax.numpy as jnp
from jax import lax
from jax.experimental import pallas as pl
from jax.experimental.pallas import tpu as pltpu
from jax.experimental.pallas import tpu_sc as plsc

_NC = 2
_NS = 16
_NW = _NC * _NS
_LANES = 16
_CHUNK = 128


def _node_mm(h, wa, n_block):
    n, nd = h.shape
    w = wa.shape[1]

    def body(h_ref, w_ref, o_ref):
        o_ref[...] = jnp.dot(h_ref[...], w_ref[...],
                             preferred_element_type=jnp.float32)

    return pl.pallas_call(
        body,
        grid=(n // n_block,),
        in_specs=[
            pl.BlockSpec((n_block, nd), lambda i: (i, 0)),
            pl.BlockSpec((nd, w), lambda i: (0, 0)),
        ],
        out_specs=pl.BlockSpec((n_block, w), lambda i: (i, 0)),
        out_shape=jax.ShapeDtypeStruct((n, w), jnp.float32),
    )(h, wa)


def _edge_mm(e, wb, beff, e_block):
    ne, ed = e.shape
    w = wb.shape[1]

    def body(e_ref, w_ref, b_ref, q_ref, eo_ref):
        eb = e_ref[...]
        q_ref[...] = jnp.dot(eb, w_ref[...],
                             preferred_element_type=jnp.float32) + b_ref[...]
        eo_ref[...] = jnp.maximum(eb, 0.0)

    return pl.pallas_call(
        body,
        grid=(ne // e_block,),
        in_specs=[
            pl.BlockSpec((e_block, ed), lambda i: (i, 0)),
            pl.BlockSpec((ed, w), lambda i: (0, 0)),
            pl.BlockSpec((1, w), lambda i: (0, 0)),
        ],
        out_specs=[
            pl.BlockSpec((e_block, w), lambda i: (i, 0)),
            pl.BlockSpec((e_block, ed), lambda i: (i, 0)),
        ],
        out_shape=[
            jax.ShapeDtypeStruct((ne, w), jnp.float32),
            jax.ShapeDtypeStruct((ne, ed), jnp.float32),
        ],
    )(e, wb, beff)


def _out_mm(p0, p1, w2e, n_block):
    n, w = p0.shape
    nd = w2e.shape[1]

    def body(a_ref, b_ref, w_ref, o_ref):
        s = a_ref[...] + b_ref[...]
        o_ref[...] = jnp.maximum(
            jnp.dot(s, w_ref[...], preferred_element_type=jnp.float32), 0.0)

    return pl.pallas_call(
        body,
        grid=(n // n_block,),
        in_specs=[
            pl.BlockSpec((n_block, w), lambda i: (i, 0)),
            pl.BlockSpec((n_block, w), lambda i: (i, 0)),
            pl.BlockSpec((w, nd), lambda i: (0, 0)),
        ],
        out_specs=pl.BlockSpec((n_block, nd), lambda i: (i, 0)),
        out_shape=jax.ShapeDtypeStruct((n, nd), jnp.float32),
    )(p0, p1, w2e)


def _sc_segment_sum(src, dst, p, q):
    n, w = p.shape
    ne = src.shape[0]
    nchunks = ne // _CHUNK
    iters = -(-nchunks // _NW)
    rz = n // _NS
    ncol = w // _LANES

    mesh = plsc.VectorSubcoreMesh(core_axis_name="c", subcore_axis_name="s")

    @functools.partial(
        pl.kernel,
        mesh=mesh,
        out_type=jax.ShapeDtypeStruct((2 * n, w), jnp.float32),
        scratch_types=[
            pltpu.VMEM((_CHUNK,), jnp.int32),
            pltpu.VMEM((_CHUNK,), jnp.int32),
            pltpu.VMEM((_CHUNK, w), jnp.float32),
            pltpu.VMEM((_CHUNK, w), jnp.float32),
            pltpu.VMEM_SHARED((n, w), jnp.float32),
            pltpu.SemaphoreType.DMA,
        ],
    )
    def sc_kernel(src_hbm, dst_hbm, p_hbm, q_hbm, out_hbm,
                  sidx, didx, prows, qrows, acc, sem):
        c = lax.axis_index("c")
        s = lax.axis_index("s")
        wid = s * _NC + c

        def zrow(j, carry):
            for cc in range(ncol):
                qrows[j, pl.ds(cc * _LANES, _LANES)] = jnp.zeros(
                    (_LANES,), jnp.float32)
            return carry
        lax.fori_loop(0, _CHUNK, zrow, 0)

        r0 = s * rz
        for k in range(rz // _CHUNK):
            pltpu.sync_copy(qrows, acc.at[pl.ds(r0 + k * _CHUNK, _CHUNK)])
        rem = rz % _CHUNK
        if rem:
            pltpu.sync_copy(qrows.at[pl.ds(0, rem)],
                            acc.at[pl.ds(r0 + (rz // _CHUNK) * _CHUNK, rem)])
        plsc.subcore_barrier()

        def body(i, carry):
            g = wid + i * _NW

            @pl.when(g < nchunks)
            def _():
                base = g * _CHUNK
                pltpu.sync_copy(src_hbm.at[pl.ds(base, _CHUNK)], sidx)
                pltpu.sync_copy(dst_hbm.at[pl.ds(base, _CHUNK)], didx)
                gcp = pltpu.async_copy(p_hbm.at[sidx], prows, sem)
                pltpu.sync_copy(q_hbm.at[pl.ds(base, _CHUNK)], qrows)
                gcp.wait()

                def row(j, rc):
                    for cc in range(ncol):
                        sl = pl.ds(cc * _LANES, _LANES)
                        prows[j, sl] = jnp.maximum(
                            prows[j, sl] + qrows[j, sl], 0.0)
                    return rc
                lax.fori_loop(0, _CHUNK, row, 0)

                pltpu.sync_copy(prows, acc.at[didx], add=True)
            return carry
        lax.fori_loop(0, iters, body, 0)
        plsc.subcore_barrier()

        ob = c * n + r0
        for k in range(rz // _CHUNK):
            pltpu.sync_copy(acc.at[pl.ds(r0 + k * _CHUNK, _CHUNK)],
                            out_hbm.at[pl.ds(ob + k * _CHUNK, _CHUNK)])
        if rem:
            pltpu.sync_copy(
                acc.at[pl.ds(r0 + (rz // _CHUNK) * _CHUNK, rem)],
                out_hbm.at[pl.ds(ob + (rz // _CHUNK) * _CHUNK, rem)])

    return sc_kernel(src, dst, p, q)


def kernel(h, edge_index, e, W1, b1, gamma, beta, run_mean, run_var, W2, b2):
    n, nd = h.shape
    ne, ed = e.shape
    emb = W1.shape[0]
    w = 160

    scale = gamma * lax.rsqrt(run_var + 1e-5)
    shift = beta - run_mean * scale
    w1s = W1 * scale[:, None]
    beff = b1 * scale + shift

    wa = jnp.zeros((nd, w), jnp.float32).at[:, :emb].set(w1s[:, :nd].T)
    wb = jnp.zeros((ed, w), jnp.float32).at[:, :emb].set(w1s[:, nd:].T)
    brow = jnp.zeros((1, w), jnp.float32).at[0, :emb].set(beff)
    brow = brow.at[0, emb].set(1.0)
    w2e = jnp.zeros((w, nd), jnp.float32).at[:emb, :].set(W2.T)
    w2e = w2e.at[emb, :].set(b2)

    src = edge_index[0].astype(jnp.int32)
    dst = edge_index[1].astype(jnp.int32)

    p = _node_mm(h, wa, n_block=1000)
    q, e_out = _edge_mm(e, wb, brow, e_block=2000)
    partials = _sc_segment_sum(src, dst, p, q)
    h_out = _out_mm(partials[:n], partials[n:], w2e, n_block=1000)
    return (h_out, e_out)


# speedup vs baseline: 1.5936x; 1.5936x over previous
import functools

import jax
import jax.numpy as jnp
from jax import lax
from jax.experimental import pallas as pl
from jax.experimental.pallas import tpu as pltpu
from jax.experimental.pallas import tpu_sc as plsc

_NC = 2
_NS = 16
_LANES = 16
_CHUNK = 128
_W = 160
_HW = _W // 2


def _node_mm(h, wa0, wa1, n_block):
    n, nd = h.shape

    def body(h_ref, w0_ref, w1_ref, o0_ref, o1_ref):
        hb = h_ref[...]
        o0_ref[...] = jnp.dot(hb, w0_ref[...],
                              preferred_element_type=jnp.float32)
        o1_ref[...] = jnp.dot(hb, w1_ref[...],
                              preferred_element_type=jnp.float32)

    return pl.pallas_call(
        body,
        grid=(n // n_block,),
        in_specs=[
            pl.BlockSpec((n_block, nd), lambda i: (i, 0)),
            pl.BlockSpec((nd, _HW), lambda i: (0, 0)),
            pl.BlockSpec((nd, _HW), lambda i: (0, 0)),
        ],
        out_specs=[
            pl.BlockSpec((n_block, _HW), lambda i: (i, 0)),
            pl.BlockSpec((n_block, _HW), lambda i: (i, 0)),
        ],
        out_shape=[
            jax.ShapeDtypeStruct((n, _HW), jnp.float32),
            jax.ShapeDtypeStruct((n, _HW), jnp.float32),
        ],
    )(h, wa0, wa1)


def _edge_mm(e, wb0, wb1, br0, br1, e_block):
    ne, ed = e.shape

    def body(e_ref, w0_ref, w1_ref, b0_ref, b1_ref, q0_ref, q1_ref, eo_ref):
        eb = e_ref[...]
        q0_ref[...] = jnp.dot(eb, w0_ref[...],
                              preferred_element_type=jnp.float32) + b0_ref[...]
        q1_ref[...] = jnp.dot(eb, w1_ref[...],
                              preferred_element_type=jnp.float32) + b1_ref[...]
        eo_ref[...] = jnp.maximum(eb, 0.0)

    return pl.pallas_call(
        body,
        grid=(ne // e_block,),
        in_specs=[
            pl.BlockSpec((e_block, ed), lambda i: (i, 0)),
            pl.BlockSpec((ed, _HW), lambda i: (0, 0)),
            pl.BlockSpec((ed, _HW), lambda i: (0, 0)),
            pl.BlockSpec((1, _HW), lambda i: (0, 0)),
            pl.BlockSpec((1, _HW), lambda i: (0, 0)),
        ],
        out_specs=[
            pl.BlockSpec((e_block, _HW), lambda i: (i, 0)),
            pl.BlockSpec((e_block, _HW), lambda i: (i, 0)),
            pl.BlockSpec((e_block, ed), lambda i: (i, 0)),
        ],
        out_shape=[
            jax.ShapeDtypeStruct((ne, _HW), jnp.float32),
            jax.ShapeDtypeStruct((ne, _HW), jnp.float32),
            jax.ShapeDtypeStruct((ne, ed), jnp.float32),
        ],
    )(e, wb0, wb1, br0, br1)


def _out_mm(s0, s1, w2top, w2bot, n_block):
    n = s0.shape[0]
    nd = w2top.shape[1]

    def body(a_ref, b_ref, wt_ref, wb_ref, o_ref):
        acc = jnp.dot(a_ref[...], wt_ref[...],
                      preferred_element_type=jnp.float32)
        acc = acc + jnp.dot(b_ref[...], wb_ref[...],
                            preferred_element_type=jnp.float32)
        o_ref[...] = jnp.maximum(acc, 0.0)

    return pl.pallas_call(
        body,
        grid=(n // n_block,),
        in_specs=[
            pl.BlockSpec((n_block, _HW), lambda i: (i, 0)),
            pl.BlockSpec((n_block, _HW), lambda i: (i, 0)),
            pl.BlockSpec((_HW, nd), lambda i: (0, 0)),
            pl.BlockSpec((_HW, nd), lambda i: (0, 0)),
        ],
        out_specs=pl.BlockSpec((n_block, nd), lambda i: (i, 0)),
        out_shape=jax.ShapeDtypeStruct((n, nd), jnp.float32),
    )(s0, s1, w2top, w2bot)


def _sc_segment_sum(src, dst, p0, p1, q0, q1):
    n = p0.shape[0]
    ne = src.shape[0]
    nchunks = ne // _CHUNK
    iters = -(-nchunks // _NS)
    rz = (n // _NS) // 8 * 8
    nextra = (n - rz * _NS) // 8
    ncol = _HW // _LANES

    mesh = plsc.VectorSubcoreMesh(core_axis_name="c", subcore_axis_name="s")

    @functools.partial(
        pl.kernel,
        mesh=mesh,
        out_type=[
            jax.ShapeDtypeStruct((n, _HW), jnp.float32),
            jax.ShapeDtypeStruct((n, _HW), jnp.float32),
        ],
        scratch_types=[
            pltpu.VMEM((_CHUNK,), jnp.int32),
            pltpu.VMEM((_CHUNK,), jnp.int32),
            pltpu.VMEM((_CHUNK, _HW), jnp.float32),
            pltpu.VMEM((_CHUNK, _HW), jnp.float32),
            pltpu.VMEM_SHARED((n, _HW), jnp.float32),
            pltpu.SemaphoreType.DMA,
        ],
        compiler_params=pltpu.CompilerParams(use_tc_tiling_on_sc=False),
    )
    def sc_kernel(src_hbm, dst_hbm, p0_hbm, p1_hbm, q0_hbm, q1_hbm,
                  out0_hbm, out1_hbm, sidx, didx, prows, qrows, acc, sem):
        c = lax.axis_index("c")
        s = lax.axis_index("s")

        def zrow(j, carry):
            for cc in range(ncol):
                qrows[j, pl.ds(cc * _LANES, _LANES)] = jnp.zeros(
                    (_LANES,), jnp.float32)
            return carry
        lax.fori_loop(0, _CHUNK, zrow, 0)

        r0 = s * rz
        rem = rz % _CHUNK
        nfull = rz // _CHUNK
        for k in range(nfull):
            pltpu.sync_copy(qrows, acc.at[pl.ds(r0 + k * _CHUNK, _CHUNK)])
        if rem:
            pltpu.sync_copy(qrows.at[pl.ds(0, rem)],
                            acc.at[pl.ds(r0 + nfull * _CHUNK, rem)])

        @pl.when(s < nextra)
        def _():
            pltpu.sync_copy(qrows.at[pl.ds(0, 8)],
                            acc.at[pl.ds(rz * _NS + s * 8, 8)])
        plsc.subcore_barrier()

        def body(i, carry):
            g = s + i * _NS

            @pl.when(g < nchunks)
            def _():
                base = g * _CHUNK
                pltpu.sync_copy(src_hbm.at[pl.ds(base, _CHUNK)], sidx)
                pltpu.sync_copy(dst_hbm.at[pl.ds(base, _CHUNK)], didx)

                @pl.when(c == 0)
                def _():
                    gcp = pltpu.async_copy(p0_hbm.at[sidx], prows, sem)
                    pltpu.sync_copy(q0_hbm.at[pl.ds(base, _CHUNK)], qrows)
                    gcp.wait()

                @pl.when(c == 1)
                def _():
                    gcp = pltpu.async_copy(p1_hbm.at[sidx], prows, sem)
                    pltpu.sync_copy(q1_hbm.at[pl.ds(base, _CHUNK)], qrows)
                    gcp.wait()

                def row(j, rc):
                    for cc in range(ncol):
                        sl = pl.ds(cc * _LANES, _LANES)
                        prows[j, sl] = jnp.maximum(
                            prows[j, sl] + qrows[j, sl], 0.0)
                    return rc
                lax.fori_loop(0, _CHUNK, row, 0)

                pltpu.sync_copy(prows, acc.at[didx], add=True)
            return carry
        lax.fori_loop(0, iters, body, 0)
        plsc.subcore_barrier()

        @pl.when(c == 0)
        def _():
            for k in range(nfull):
                pltpu.sync_copy(acc.at[pl.ds(r0 + k * _CHUNK, _CHUNK)],
                                out0_hbm.at[pl.ds(r0 + k * _CHUNK, _CHUNK)])
            if rem:
                pltpu.sync_copy(acc.at[pl.ds(r0 + nfull * _CHUNK, rem)],
                                out0_hbm.at[pl.ds(r0 + nfull * _CHUNK, rem)])

            @pl.when(s < nextra)
            def _():
                pltpu.sync_copy(acc.at[pl.ds(rz * _NS + s * 8, 8)],
                                out0_hbm.at[pl.ds(rz * _NS + s * 8, 8)])

        @pl.when(c == 1)
        def _():
            for k in range(nfull):
                pltpu.sync_copy(acc.at[pl.ds(r0 + k * _CHUNK, _CHUNK)],
                                out1_hbm.at[pl.ds(r0 + k * _CHUNK, _CHUNK)])
            if rem:
                pltpu.sync_copy(acc.at[pl.ds(r0 + nfull * _CHUNK, rem)],
                                out1_hbm.at[pl.ds(r0 + nfull * _CHUNK, rem)])

            @pl.when(s < nextra)
            def _():
                pltpu.sync_copy(acc.at[pl.ds(rz * _NS + s * 8, 8)],
                                out1_hbm.at[pl.ds(rz * _NS + s * 8, 8)])

    return sc_kernel(src, dst, p0, p1, q0, q1)


def kernel(h, edge_index, e, W1, b1, gamma, beta, run_mean, run_var, W2, b2):
    n, nd = h.shape
    ne, ed = e.shape
    emb = W1.shape[0]

    scale = gamma * lax.rsqrt(run_var + 1e-5)
    shift = beta - run_mean * scale
    w1s = W1 * scale[:, None]
    beff = b1 * scale + shift

    wa = jnp.zeros((nd, _W), jnp.float32).at[:, :emb].set(w1s[:, :nd].T)
    wb = jnp.zeros((ed, _W), jnp.float32).at[:, :emb].set(w1s[:, nd:].T)
    brow = jnp.zeros((1, _W), jnp.float32).at[0, :emb].set(beff)
    brow = brow.at[0, emb].set(1.0)
    w2e = jnp.zeros((_W, nd), jnp.float32).at[:emb, :].set(W2.T)
    w2e = w2e.at[emb, :].set(b2)

    src = edge_index[0].astype(jnp.int32)
    dst = edge_index[1].astype(jnp.int32)

    p0, p1 = _node_mm(h, wa[:, :_HW], wa[:, _HW:], n_block=1000)
    q0, q1, e_out = _edge_mm(e, wb[:, :_HW], wb[:, _HW:],
                             brow[:, :_HW], brow[:, _HW:], e_block=2000)
    s0, s1 = _sc_segment_sum(src, dst, p0, p1, q0, q1)
    h_out = _out_mm(s0, s1, w2e[:_HW], w2e[_HW:], n_block=1000)
    return (h_out, e_out)
